# Initial kernel scaffold; baseline (speedup 1.0000x reference)
import jax
import jax.numpy as jnp
from jax.experimental import pallas as pl


def kernel(audio_emb, text_emb, labels):
    raise NotImplementedError("write your pallas kernel here")


# speedup vs baseline: 1.6232x; 1.6232x over previous
import jax
import jax.numpy as jnp
from jax.experimental import pallas as pl
from jax.experimental.pallas import tpu as pltpu

_EPSILON = 0.05
_REG = 0.1
_NUM_ITER = 10
_B = 4096
_D = 1024
_R_MM = 256
_R = 128


def _wloss_kernel(a_ref, tT_ref, out_ref, k_ref, u_ref, v_ref):
    fB = jnp.float32(1.0 / _B)

    tsq = jnp.sum(tT_ref[...].astype(jnp.float32) ** 2, axis=0, keepdims=True)
    rt = jax.lax.rsqrt(tsq)

    def mm_body(c, gmin):
        ac = a_ref[pl.ds(c * _R_MM, _R_MM), :]
        acf = ac.astype(jnp.float32)
        ra = jax.lax.rsqrt(jnp.sum(acf * acf, axis=1, keepdims=True))
        g = jax.lax.dot_general(
            ac, tT_ref[...],
            dimension_numbers=(((1,), (0,)), ((), ())),
            preferred_element_type=jnp.float32)
        g = g * ra * rt
        k_ref[pl.ds(c * _R_MM, _R_MM), :] = g.astype(jnp.bfloat16)
        return jnp.minimum(gmin, jnp.min(g))

    gmin = jax.lax.fori_loop(0, _B // _R_MM, mm_body, jnp.float32(2.0))

    cexp = 1.0 / (_EPSILON * (1.0 - gmin))

    def exp_body(c, carry):
        g = k_ref[pl.ds(c * _R, _R), :].astype(jnp.float32)
        k_ref[pl.ds(c * _R, _R), :] = jnp.exp((g - 1.0) * cexp).astype(jnp.bfloat16)
        return carry

    jax.lax.fori_loop(0, _B // _R, exp_body, 0)

    u_ref[...] = jnp.full((_B, 1), fB, jnp.float32)

    def col_pass(_c, acc):
        k = k_ref[pl.ds(_c * _R, _R), :].astype(jnp.float32)
        uc = u_ref[pl.ds(_c * _R, _R), :]
        return acc + jnp.sum(k * uc, axis=0, keepdims=True)

    def sink_body(_it, carry):
        ktu = jax.lax.fori_loop(0, _B // _R, col_pass,
                                jnp.zeros((1, _B), jnp.float32))
        v_ref[...] = fB / ktu

        def row_pass(c, inner):
            k = k_ref[pl.ds(c * _R, _R), :].astype(jnp.float32)
            kv = jnp.sum(k * v_ref[...], axis=1, keepdims=True)
            u_ref[pl.ds(c * _R, _R), :] = fB / kv
            return inner

        jax.lax.fori_loop(0, _B // _R, row_pass, 0)
        return carry

    jax.lax.fori_loop(0, _NUM_ITER, sink_body, 0)

    ktu = jax.lax.fori_loop(0, _B // _R, col_pass,
                            jnp.zeros((1, _B), jnp.float32))
    col = v_ref[...] * ktu
    klc = jnp.where(col > 0, col * (jnp.log(col) - fB), 0.0)
    kl_col = jnp.sum(klc) * fB

    def ce_pass(c, accs):
        ce_acc, klr_acc = accs
        k = k_ref[pl.ds(c * _R, _R), :].astype(jnp.float32)
        t = k * v_ref[...]
        kv = jnp.sum(t, axis=1, keepdims=True)
        uc = u_ref[pl.ds(c * _R, _R), :]
        row = uc * kv
        klr = jnp.where(row > 0, row * (jnp.log(row) - fB), 0.0)
        klr_acc = klr_acc + jnp.sum(klr)
        p = uc * t
        m = jnp.max(p, axis=1, keepdims=True)
        s = jnp.sum(jnp.exp(p - m), axis=1, keepdims=True)
        lse = m + jnp.log(s)
        lane = jax.lax.broadcasted_iota(jnp.int32, (_R, _B), 1)
        rid = jax.lax.broadcasted_iota(jnp.int32, (_R, _B), 0) + c * _R
        d = jnp.sum(jnp.where(lane == rid, p, 0.0), axis=1, keepdims=True)
        ce_acc = ce_acc + jnp.sum(lse - d)
        return (ce_acc, klr_acc)

    ce_sum, klr_sum = jax.lax.fori_loop(
        0, _B // _R, ce_pass, (jnp.float32(0.0), jnp.float32(0.0)))
    ce = ce_sum * fB
    kl_row = klr_sum * fB
    out_ref[0, 0] = ce + _REG * (kl_col + kl_row)


def kernel(audio_emb, text_emb, labels):
    del labels
    a = audio_emb.astype(jnp.bfloat16)
    tT = text_emb.astype(jnp.bfloat16).T
    out = pl.pallas_call(
        _wloss_kernel,
        out_shape=jax.ShapeDtypeStruct((1, 1), jnp.float32),
        in_specs=[pl.BlockSpec(memory_space=pltpu.VMEM),
                  pl.BlockSpec(memory_space=pltpu.VMEM)],
        out_specs=pl.BlockSpec(memory_space=pltpu.SMEM),
        scratch_shapes=[
            pltpu.VMEM((_B, _B), jnp.bfloat16),
            pltpu.VMEM((_B, 1), jnp.float32),
            pltpu.VMEM((1, _B), jnp.float32),
        ],
        compiler_params=pltpu.CompilerParams(
            vmem_limit_bytes=100 * 1024 * 1024),
    )(a, tT)
    return out[0, 0]


# speedup vs baseline: 2.6386x; 1.6256x over previous
import jax
import jax.numpy as jnp
from jax.experimental import pallas as pl
from jax.experimental.pallas import tpu as pltpu

_EPSILON = 0.05
_REG = 0.1
_NUM_ITER = 10
_B = 4096
_D = 1024
_R_MM = 256
_R = 256
_NC = _B // _R


def _bf16_sum0(x):
    rows = x.shape[0]
    while rows > 16:
        rows //= 2
        x = x[:rows, :] + x[rows:2 * rows, :]
    return jnp.sum(x.astype(jnp.float32), axis=0, keepdims=True)


def _bf16_sum1(x):
    cols = x.shape[1]
    while cols > 256:
        cols //= 2
        x = x[:, :cols] + x[:, cols:2 * cols]
    return jnp.sum(x.astype(jnp.float32), axis=1, keepdims=True)


def _wloss_kernel(a_ref, tT_ref, out_ref, k_ref, u_ref, v_ref):
    fB = jnp.float32(1.0 / _B)

    tsq = jnp.sum(tT_ref[...].astype(jnp.float32) ** 2, axis=0, keepdims=True)
    rt = jax.lax.rsqrt(tsq)

    def mm_body(c, gmin):
        ac = a_ref[pl.ds(c * _R_MM, _R_MM), :]
        acf = ac.astype(jnp.float32)
        ra = jax.lax.rsqrt(jnp.sum(acf * acf, axis=1, keepdims=True))
        g = jax.lax.dot_general(
            ac, tT_ref[...],
            dimension_numbers=(((1,), (0,)), ((), ())),
            preferred_element_type=jnp.float32)
        g = g * ra * rt
        k_ref[pl.ds(c * _R_MM, _R_MM), :] = g.astype(jnp.bfloat16)
        return jnp.minimum(gmin, jnp.min(g))

    gmin = jax.lax.fori_loop(0, _B // _R_MM, mm_body, jnp.float32(2.0))

    cexp = 1.0 / (_EPSILON * (1.0 - gmin))

    def exp_body(c, acc):
        g = k_ref[pl.ds(c * _R, _R), :].astype(jnp.float32)
        e = jnp.exp((g - 1.0) * cexp).astype(jnp.bfloat16)
        k_ref[pl.ds(c * _R, _R), :] = e
        return acc + _bf16_sum0(e)

    csum = jax.lax.fori_loop(0, _NC, exp_body,
                             jnp.zeros((1, _B), jnp.float32))

    def bf_row_pass(c, carry):
        k = k_ref[pl.ds(c * _R, _R), :]
        kv = _bf16_sum1(k * v_ref[...])
        u_ref[pl.ds(c * _R, _R), :] = (fB / kv).astype(jnp.bfloat16)
        return carry

    def bf_col_pass(c, acc):
        k = k_ref[pl.ds(c * _R, _R), :]
        uc = u_ref[pl.ds(c * _R, _R), :]
        return acc + _bf16_sum0(k * uc)

    v_ref[...] = (1.0 / csum).astype(jnp.bfloat16)
    jax.lax.fori_loop(0, _NC, bf_row_pass, 0)

    def sink_body(_it, carry):
        ktu = jax.lax.fori_loop(0, _NC, bf_col_pass,
                                jnp.zeros((1, _B), jnp.float32))
        v_ref[...] = (fB / ktu).astype(jnp.bfloat16)
        jax.lax.fori_loop(0, _NC, bf_row_pass, 0)
        return carry

    jax.lax.fori_loop(0, _NUM_ITER - 2, sink_body, 0)

    ktu = jax.lax.fori_loop(0, _NC, bf_col_pass,
                            jnp.zeros((1, _B), jnp.float32))
    v_ref[...] = (fB / ktu).astype(jnp.bfloat16)

    def ce_pass(c, accs):
        ce_acc, klr_acc = accs
        k = k_ref[pl.ds(c * _R, _R), :].astype(jnp.float32)
        t = k * v_ref[...].astype(jnp.float32)
        kv = jnp.sum(t, axis=1, keepdims=True)
        u = fB / kv
        u_ref[pl.ds(c * _R, _R), :] = u.astype(jnp.bfloat16)
        row = u * kv
        klr = jnp.where(row > 0, row * (jnp.log(row) - fB), 0.0)
        klr_acc = klr_acc + jnp.sum(klr)
        p = u * t
        m = jnp.max(p, axis=1, keepdims=True)
        s = jnp.sum(jnp.exp(p - m), axis=1, keepdims=True)
        lse = m + jnp.log(s)
        lane = jax.lax.broadcasted_iota(jnp.int32, (_R, _B), 1)
        rid = jax.lax.broadcasted_iota(jnp.int32, (_R, _B), 0) + c * _R
        d = jnp.sum(jnp.where(lane == rid, p, 0.0), axis=1, keepdims=True)
        ce_acc = ce_acc + jnp.sum(lse - d)
        return (ce_acc, klr_acc)

    ce_sum, klr_sum = jax.lax.fori_loop(
        0, _NC, ce_pass, (jnp.float32(0.0), jnp.float32(0.0)))
    ce = ce_sum * fB
    kl_row = klr_sum * fB

    def col_pass_f32(c, acc):
        k = k_ref[pl.ds(c * _R, _R), :].astype(jnp.float32)
        uc = u_ref[pl.ds(c * _R, _R), :].astype(jnp.float32)
        return acc + jnp.sum(k * uc, axis=0, keepdims=True)

    ktu_f = jax.lax.fori_loop(0, _NC, col_pass_f32,
                              jnp.zeros((1, _B), jnp.float32))
    col = v_ref[...].astype(jnp.float32) * ktu_f
    klc = jnp.where(col > 0, col * (jnp.log(col) - fB), 0.0)
    kl_col = jnp.sum(klc) * fB

    out_ref[0, 0] = ce + _REG * (kl_col + kl_row)


def kernel(audio_emb, text_emb, labels):
    del labels
    a = audio_emb.astype(jnp.bfloat16)
    tT = text_emb.astype(jnp.bfloat16).T
    out = pl.pallas_call(
        _wloss_kernel,
        out_shape=jax.ShapeDtypeStruct((1, 1), jnp.float32),
        in_specs=[pl.BlockSpec(memory_space=pltpu.VMEM),
                  pl.BlockSpec(memory_space=pltpu.VMEM)],
        out_specs=pl.BlockSpec(memory_space=pltpu.SMEM),
        scratch_shapes=[
            pltpu.VMEM((_B, _B), jnp.bfloat16),
            pltpu.VMEM((_B, 1), jnp.bfloat16),
            pltpu.VMEM((1, _B), jnp.bfloat16),
        ],
        compiler_params=pltpu.CompilerParams(
            vmem_limit_bytes=100 * 1024 * 1024),
    )(a, tT)
    return out[0, 0]


# speedup vs baseline: 3.0511x; 1.1563x over previous
import jax
import jax.numpy as jnp
from jax.experimental import pallas as pl
from jax.experimental.pallas import tpu as pltpu

_EPSILON = 0.05
_REG = 0.1
_NUM_ITER = 10
_B = 4096
_D = 1024
_R_MM = 256
_R = 256
_NC = _B // _R


def _bf16_sum0(x):
    rows = x.shape[0]
    while rows > 16:
        rows //= 2
        x = x[:rows, :] + x[rows:2 * rows, :]
    return jnp.sum(x.astype(jnp.float32), axis=0, keepdims=True)


def _bf16_sum1(x):
    cols = x.shape[1]
    while cols > 256:
        cols //= 2
        x = x[:, :cols] + x[:, cols:2 * cols]
    return jnp.sum(x.astype(jnp.float32), axis=1, keepdims=True)


def _wloss_kernel(a_ref, tT_ref, out_ref, k_ref, u_ref, v_ref):
    fB = jnp.float32(1.0 / _B)

    tsq = jnp.sum(tT_ref[...].astype(jnp.float32) ** 2, axis=0, keepdims=True)
    rt = jax.lax.rsqrt(tsq)

    def mm_body(c, gmin):
        ac = a_ref[pl.ds(c * _R_MM, _R_MM), :]
        acf = ac.astype(jnp.float32)
        ra = jax.lax.rsqrt(jnp.sum(acf * acf, axis=1, keepdims=True))
        g = jax.lax.dot_general(
            ac, tT_ref[...],
            dimension_numbers=(((1,), (0,)), ((), ())),
            preferred_element_type=jnp.float32)
        g = g * ra * rt
        k_ref[pl.ds(c * _R_MM, _R_MM), :] = g.astype(jnp.bfloat16)
        return jnp.minimum(gmin, jnp.min(g))

    gmin = jax.lax.fori_loop(0, _B // _R_MM, mm_body, jnp.float32(2.0))

    cexp = 1.0 / (_EPSILON * (1.0 - gmin))

    def exp_body(c, acc):
        g = k_ref[pl.ds(c * _R, _R), :].astype(jnp.float32)
        e = jnp.exp((g - 1.0) * cexp).astype(jnp.bfloat16)
        k_ref[pl.ds(c * _R, _R), :] = e
        return acc + _bf16_sum0(e)

    csum = jax.lax.fori_loop(0, _NC, exp_body,
                             jnp.zeros((1, _B), jnp.float32))

    def bf_row_pass(c, carry):
        k = k_ref[pl.ds(c * _R, _R), :]
        kv = _bf16_sum1(k * v_ref[...])
        u_ref[pl.ds(c * _R, _R), :] = (fB / kv).astype(jnp.bfloat16)
        return carry

    def bf_col_pass(c, acc):
        k = k_ref[pl.ds(c * _R, _R), :]
        uc = u_ref[pl.ds(c * _R, _R), :]
        return acc + _bf16_sum0(k * uc)

    v_ref[...] = (1.0 / csum).astype(jnp.bfloat16)
    jax.lax.fori_loop(0, _NC, bf_row_pass, 0)

    def sink_body(_it, carry):
        ktu = jax.lax.fori_loop(0, _NC, bf_col_pass,
                                jnp.zeros((1, _B), jnp.float32))
        v_ref[...] = (fB / ktu).astype(jnp.bfloat16)
        jax.lax.fori_loop(0, _NC, bf_row_pass, 0)
        return carry

    jax.lax.fori_loop(0, _NUM_ITER - 2, sink_body, 0)

    ktu = jax.lax.fori_loop(0, _NC, bf_col_pass,
                            jnp.zeros((1, _B), jnp.float32))
    v_ref[...] = (fB / ktu).astype(jnp.bfloat16)

    def ce_pass(c, accs):
        ce_acc, klr_acc = accs
        k = k_ref[pl.ds(c * _R, _R), :]
        kv = _bf16_sum1(k * v_ref[...])
        u = fB / kv
        u_ref[pl.ds(c * _R, _R), :] = u.astype(jnp.bfloat16)
        row = u * kv
        klr = jnp.where(row > 0, row * (jnp.log(row) - fB), 0.0)
        klr_acc = klr_acc + jnp.sum(klr)
        off = pl.multiple_of(c * _R, _R)
        kd = k_ref[pl.ds(off, _R), pl.ds(off, _R)]
        vd = v_ref[:, pl.ds(off, _R)]
        tb = (kd * vd).astype(jnp.float32)
        eye = (jax.lax.broadcasted_iota(jnp.int32, (_R, _R), 0)
               == jax.lax.broadcasted_iota(jnp.int32, (_R, _R), 1))
        d = jnp.sum(jnp.where(eye, tb, 0.0), axis=1, keepdims=True)
        lse = jnp.log(jnp.float32(_B) + row)
        ce_acc = ce_acc + jnp.sum(lse - u * d)
        return (ce_acc, klr_acc)

    ce_sum, klr_sum = jax.lax.fori_loop(
        0, _NC, ce_pass, (jnp.float32(0.0), jnp.float32(0.0)))
    ce = ce_sum * fB
    kl_row = klr_sum * fB

    ktu_f = jax.lax.fori_loop(0, _NC, bf_col_pass,
                              jnp.zeros((1, _B), jnp.float32))
    col = v_ref[...].astype(jnp.float32) * ktu_f
    klc = jnp.where(col > 0, col * (jnp.log(col) - fB), 0.0)
    kl_col = jnp.sum(klc) * fB

    out_ref[0, 0] = ce + _REG * (kl_col + kl_row)


def kernel(audio_emb, text_emb, labels):
    del labels
    a = audio_emb.astype(jnp.bfloat16)
    tT = text_emb.astype(jnp.bfloat16).T
    out = pl.pallas_call(
        _wloss_kernel,
        out_shape=jax.ShapeDtypeStruct((1, 1), jnp.float32),
        in_specs=[pl.BlockSpec(memory_space=pltpu.VMEM),
                  pl.BlockSpec(memory_space=pltpu.VMEM)],
        out_specs=pl.BlockSpec(memory_space=pltpu.SMEM),
        scratch_shapes=[
            pltpu.VMEM((_B, _B), jnp.bfloat16),
            pltpu.VMEM((_B, 1), jnp.bfloat16),
            pltpu.VMEM((1, _B), jnp.bfloat16),
        ],
        compiler_params=pltpu.CompilerParams(
            vmem_limit_bytes=100 * 1024 * 1024),
    )(a, tT)
    return out[0, 0]


# speedup vs baseline: 3.3537x; 1.0992x over previous
import jax
import jax.numpy as jnp
from jax.experimental import pallas as pl
from jax.experimental.pallas import tpu as pltpu

_EPSILON = 0.05
_REG = 0.1
_NUM_ITER = 10
_B = 4096
_D = 1024
_R_MM = 256
_R = 256
_NC = _B // _R


def _bf16_sum0(x):
    rows = x.shape[0]
    while rows > 16:
        rows //= 2
        x = x[:rows, :] + x[rows:2 * rows, :]
    return jnp.sum(x.astype(jnp.float32), axis=0, keepdims=True)


def _bf16_sum1(x):
    cols = x.shape[1]
    while cols > 256:
        cols //= 2
        x = x[:, :cols] + x[:, cols:2 * cols]
    return jnp.sum(x.astype(jnp.float32), axis=1, keepdims=True)


def _wloss_kernel(a_ref, tT_ref, out_ref, k_ref, u_ref, v_ref):
    fB = jnp.float32(1.0 / _B)

    tsq = jnp.sum(tT_ref[...].astype(jnp.float32) ** 2, axis=0, keepdims=True)
    rt = jax.lax.rsqrt(tsq)

    def mm_body(c, gmin):
        ac = a_ref[pl.ds(c * _R_MM, _R_MM), :]
        acf = ac.astype(jnp.float32)
        ra = jax.lax.rsqrt(jnp.sum(acf * acf, axis=1, keepdims=True))
        g = jax.lax.dot_general(
            ac, tT_ref[...],
            dimension_numbers=(((1,), (0,)), ((), ())),
            preferred_element_type=jnp.float32)
        g = g * ra * rt
        k_ref[pl.ds(c * _R_MM, _R_MM), :] = g.astype(jnp.bfloat16)
        return jnp.minimum(gmin, jnp.min(g))

    gmin = jax.lax.fori_loop(0, _B // _R_MM, mm_body, jnp.float32(2.0))

    cexp = 1.0 / (_EPSILON * (1.0 - gmin))

    def exp_body(c, acc):
        g = k_ref[pl.ds(c * _R, _R), :].astype(jnp.float32)
        e = jnp.exp((g - 1.0) * cexp).astype(jnp.bfloat16)
        k_ref[pl.ds(c * _R, _R), :] = e
        return acc + _bf16_sum0(e)

    csum = jax.lax.fori_loop(0, _NC, exp_body,
                             jnp.zeros((1, _B), jnp.float32))

    def bf_col_pass(c, acc):
        k = k_ref[pl.ds(c * _R, _R), :]
        uc = u_ref[pl.ds(c * _R, _R), :]
        return acc + _bf16_sum0(k * uc)

    def rowcol_scan(c, acc):
        k = k_ref[pl.ds(c * _R, _R), :]
        kv = _bf16_sum1(k * v_ref[...])
        u = (fB / kv).astype(jnp.bfloat16)
        return acc + _bf16_sum0(k * u)

    v_ref[...] = (1.0 / csum).astype(jnp.bfloat16)

    def sink_body(_it, carry):
        ktu = jax.lax.fori_loop(0, _NC, rowcol_scan,
                                jnp.zeros((1, _B), jnp.float32))
        v_ref[...] = (fB / ktu).astype(jnp.bfloat16)
        return carry

    jax.lax.fori_loop(0, _NUM_ITER - 1, sink_body, 0)

    def ce_pass(c, accs):
        ce_acc, klr_acc = accs
        k = k_ref[pl.ds(c * _R, _R), :]
        kv = _bf16_sum1(k * v_ref[...])
        u = fB / kv
        u_ref[pl.ds(c * _R, _R), :] = u.astype(jnp.bfloat16)
        row = u * kv
        klr = jnp.where(row > 0, row * (jnp.log(row) - fB), 0.0)
        klr_acc = klr_acc + jnp.sum(klr)
        off = pl.multiple_of(c * _R, _R)
        kd = k_ref[pl.ds(off, _R), pl.ds(off, _R)]
        vd = v_ref[:, pl.ds(off, _R)]
        tb = (kd * vd).astype(jnp.float32)
        eye = (jax.lax.broadcasted_iota(jnp.int32, (_R, _R), 0)
               == jax.lax.broadcasted_iota(jnp.int32, (_R, _R), 1))
        d = jnp.sum(jnp.where(eye, tb, 0.0), axis=1, keepdims=True)
        lse = jnp.log(jnp.float32(_B) + row)
        ce_acc = ce_acc + jnp.sum(lse - u * d)
        return (ce_acc, klr_acc)

    ce_sum, klr_sum = jax.lax.fori_loop(
        0, _NC, ce_pass, (jnp.float32(0.0), jnp.float32(0.0)))
    ce = ce_sum * fB
    kl_row = klr_sum * fB

    ktu_f = jax.lax.fori_loop(0, _NC, bf_col_pass,
                              jnp.zeros((1, _B), jnp.float32))
    col = v_ref[...].astype(jnp.float32) * ktu_f
    klc = jnp.where(col > 0, col * (jnp.log(col) - fB), 0.0)
    kl_col = jnp.sum(klc) * fB

    out_ref[0, 0] = ce + _REG * (kl_col + kl_row)


def kernel(audio_emb, text_emb, labels):
    del labels
    a = audio_emb.astype(jnp.bfloat16)
    tT = text_emb.astype(jnp.bfloat16).T
    out = pl.pallas_call(
        _wloss_kernel,
        out_shape=jax.ShapeDtypeStruct((1, 1), jnp.float32),
        in_specs=[pl.BlockSpec(memory_space=pltpu.VMEM),
                  pl.BlockSpec(memory_space=pltpu.VMEM)],
        out_specs=pl.BlockSpec(memory_space=pltpu.SMEM),
        scratch_shapes=[
            pltpu.VMEM((_B, _B), jnp.bfloat16),
            pltpu.VMEM((_B, 1), jnp.bfloat16),
            pltpu.VMEM((1, _B), jnp.bfloat16),
        ],
        compiler_params=pltpu.CompilerParams(
            vmem_limit_bytes=100 * 1024 * 1024),
    )(a, tT)
    return out[0, 0]


# speedup vs baseline: 3.5383x; 1.0551x over previous
import jax
import jax.numpy as jnp
from jax.experimental import pallas as pl
from jax.experimental.pallas import tpu as pltpu

_EPSILON = 0.05
_REG = 0.1
_NUM_ITER = 10
_B = 4096
_D = 1024
_R_MM = 256
_R = 512
_NC = _B // _R


def _bf16_sum0(x):
    rows = x.shape[0]
    while rows > 16:
        rows //= 2
        x = x[:rows, :] + x[rows:2 * rows, :]
    return jnp.sum(x.astype(jnp.float32), axis=0, keepdims=True)


def _bf16_sum1(x):
    cols = x.shape[1]
    while cols > 256:
        cols //= 2
        x = x[:, :cols] + x[:, cols:2 * cols]
    return jnp.sum(x.astype(jnp.float32), axis=1, keepdims=True)


def _wloss_kernel(a_ref, tT_ref, out_ref, k_ref, u_ref, v_ref):
    fB = jnp.float32(1.0 / _B)

    tsq = jnp.sum(tT_ref[...].astype(jnp.float32) ** 2, axis=0, keepdims=True)
    rt = jax.lax.rsqrt(tsq)

    def mm_body(c, gmin):
        ac = a_ref[pl.ds(c * _R_MM, _R_MM), :]
        acf = ac.astype(jnp.float32)
        ra = jax.lax.rsqrt(jnp.sum(acf * acf, axis=1, keepdims=True))
        g = jax.lax.dot_general(
            ac, tT_ref[...],
            dimension_numbers=(((1,), (0,)), ((), ())),
            preferred_element_type=jnp.float32)
        g = g * ra * rt
        k_ref[pl.ds(c * _R_MM, _R_MM), :] = g.astype(jnp.bfloat16)
        return jnp.minimum(gmin, jnp.min(g))

    gmin = jax.lax.fori_loop(0, _B // _R_MM, mm_body, jnp.float32(2.0))

    cexp2 = 1.4426950408889634 / (_EPSILON * (1.0 - gmin))

    def exp_body(c, acc):
        g = k_ref[pl.ds(c * _R, _R), :].astype(jnp.float32)
        e = jnp.exp2(g * cexp2).astype(jnp.bfloat16)
        k_ref[pl.ds(c * _R, _R), :] = e
        return acc + _bf16_sum0(e)

    csum = jax.lax.fori_loop(0, _NC, exp_body,
                             jnp.zeros((1, _B), jnp.float32))

    def bf_col_pass(c, acc):
        k = k_ref[pl.ds(c * _R, _R), :]
        uc = u_ref[pl.ds(c * _R, _R), :]
        return acc + _bf16_sum0(k * uc)

    def rowcol_scan(c, acc):
        k = k_ref[pl.ds(c * _R, _R), :]
        kv = _bf16_sum1(k * v_ref[...])
        u = (fB / kv).astype(jnp.bfloat16)
        return acc + _bf16_sum0(k * u)

    v_ref[...] = (1.0 / csum).astype(jnp.bfloat16)

    def sink_body(_it, carry):
        ktu = jax.lax.fori_loop(0, _NC, rowcol_scan,
                                jnp.zeros((1, _B), jnp.float32))
        v_ref[...] = (fB / ktu).astype(jnp.bfloat16)
        return carry

    jax.lax.fori_loop(0, _NUM_ITER - 1, sink_body, 0)

    def ce_pass(c, accs):
        ce_acc, klr_acc = accs
        k = k_ref[pl.ds(c * _R, _R), :]
        kv = _bf16_sum1(k * v_ref[...])
        u = fB / kv
        u_ref[pl.ds(c * _R, _R), :] = u.astype(jnp.bfloat16)
        row = u * kv
        klr = jnp.where(row > 0, row * (jnp.log(row) - fB), 0.0)
        klr_acc = klr_acc + jnp.sum(klr)
        off = pl.multiple_of(c * _R, _R)
        kd = k_ref[pl.ds(off, _R), pl.ds(off, _R)]
        vd = v_ref[:, pl.ds(off, _R)]
        tb = (kd * vd).astype(jnp.float32)
        eye = (jax.lax.broadcasted_iota(jnp.int32, (_R, _R), 0)
               == jax.lax.broadcasted_iota(jnp.int32, (_R, _R), 1))
        d = jnp.sum(jnp.where(eye, tb, 0.0), axis=1, keepdims=True)
        lse = jnp.log(jnp.float32(_B) + row)
        ce_acc = ce_acc + jnp.sum(lse - u * d)
        return (ce_acc, klr_acc)

    ce_sum, klr_sum = jax.lax.fori_loop(
        0, _NC, ce_pass, (jnp.float32(0.0), jnp.float32(0.0)))
    ce = ce_sum * fB
    kl_row = klr_sum * fB

    ktu_f = jax.lax.fori_loop(0, _NC, bf_col_pass,
                              jnp.zeros((1, _B), jnp.float32))
    col = v_ref[...].astype(jnp.float32) * ktu_f
    klc = jnp.where(col > 0, col * (jnp.log(col) - fB), 0.0)
    kl_col = jnp.sum(klc) * fB

    out_ref[0, 0] = ce + _REG * (kl_col + kl_row)


def kernel(audio_emb, text_emb, labels):
    del labels
    a = audio_emb.astype(jnp.bfloat16)
    tT = text_emb.astype(jnp.bfloat16).T
    out = pl.pallas_call(
        _wloss_kernel,
        out_shape=jax.ShapeDtypeStruct((1, 1), jnp.float32),
        in_specs=[pl.BlockSpec(memory_space=pltpu.VMEM),
                  pl.BlockSpec(memory_space=pltpu.VMEM)],
        out_specs=pl.BlockSpec(memory_space=pltpu.SMEM),
        scratch_shapes=[
            pltpu.VMEM((_B, _B), jnp.bfloat16),
            pltpu.VMEM((_B, 1), jnp.bfloat16),
            pltpu.VMEM((1, _B), jnp.bfloat16),
        ],
        compiler_params=pltpu.CompilerParams(
            vmem_limit_bytes=100 * 1024 * 1024),
    )(a, tT)
    return out[0, 0]
